# baseline (speedup 1.0000x reference)
import functools

import jax
import jax.numpy as jnp
from jax.experimental import pallas as pl


def _gelu_exact(v):
    return 0.5 * v * (1.0 + jax.lax.erf(v * 0.7071067811865476))


def _fused_body(x_ref, pi_ref, W1_ref, b1_ref, W2_ref, b2_ref, Wd_ref, Wu_ref,
                o_ref, *, tm, kh, hid, r, lanes):
    f32 = jnp.float32
    x = x_ref[...]

    eidx = (jax.lax.broadcasted_iota(jnp.int32, (tm, lanes), 1) // r).astype(f32)
    p0 = pi_ref[:, 0:1]
    p1 = pi_ref[:, 1:2]
    i0 = pi_ref[:, 2:3]
    i1 = pi_ref[:, 3:4]
    w = (p0 * (i0 == eidx).astype(f32) + p1 * (i1 == eidx).astype(f32))

    h2 = _gelu_exact(jnp.dot(x, Wd_ref[...], preferred_element_type=f32))
    acc = jnp.dot((h2 * w).astype(jnp.bfloat16), Wu_ref[...],
                  preferred_element_type=f32)
    acc = acc + b2_ref[...]

    for k in range(hid // kh):
        h = _gelu_exact(
            jnp.dot(x, W1_ref[:, k * kh:(k + 1) * kh],
                    preferred_element_type=f32) + b1_ref[:, k * kh:(k + 1) * kh])
        acc = acc + jnp.dot(h.astype(jnp.bfloat16), W2_ref[k * kh:(k + 1) * kh, :],
                            preferred_element_type=f32)
    o_ref[...] = acc


def kernel(x, gate, topk_probs, topk_idx, W1, b1, W2, b2, w_down, w_up):
    del gate
    orig_shape = x.shape
    dim = x.shape[-1]
    e, _, r = w_down.shape
    hid = W1.shape[1]
    alpha = 8.0
    scaling = alpha / r

    xf = x.reshape(-1, dim)
    t = xf.shape[0]
    tm = min(512, t)
    kh = min(512, hid)
    lanes = 128

    wd = jnp.transpose(w_down, (1, 0, 2)).reshape(dim, e * r)
    wd = jnp.pad(wd, ((0, 0), (0, lanes - e * r))).astype(jnp.bfloat16)
    wu = w_up.reshape(e * r, dim) * scaling
    wu = jnp.pad(wu, ((0, lanes - e * r), (0, 0))).astype(jnp.bfloat16)

    pi = jnp.concatenate(
        [topk_probs.astype(jnp.float32), topk_idx.astype(jnp.float32)], axis=1)

    b1r = b1.reshape(1, hid)
    b2r = b2.reshape(1, dim)

    grid = (t // tm,)
    out = pl.pallas_call(
        functools.partial(_fused_body, tm=tm, kh=kh, hid=hid, r=r,
                          lanes=lanes),
        grid=grid,
        in_specs=[
            pl.BlockSpec((tm, dim), lambda i: (i, 0)),
            pl.BlockSpec((tm, 4), lambda i: (i, 0)),
            pl.BlockSpec((dim, hid), lambda i: (0, 0)),
            pl.BlockSpec((1, hid), lambda i: (0, 0)),
            pl.BlockSpec((hid, dim), lambda i: (0, 0)),
            pl.BlockSpec((1, dim), lambda i: (0, 0)),
            pl.BlockSpec((dim, lanes), lambda i: (0, 0)),
            pl.BlockSpec((lanes, dim), lambda i: (0, 0)),
        ],
        out_specs=pl.BlockSpec((tm, dim), lambda i: (i, 0)),
        out_shape=jax.ShapeDtypeStruct((t, dim), jnp.float32),
    )(xf.astype(jnp.bfloat16), pi, W1.astype(jnp.bfloat16), b1r,
      W2.astype(jnp.bfloat16), b2r, wd, wu)
    return out.reshape(orig_shape)


# speedup vs baseline: 1.1470x; 1.1470x over previous
import functools

import jax
import jax.numpy as jnp
from jax.experimental import pallas as pl


def _gelu_exact(v):
    return 0.5 * v * (1.0 + jax.lax.erf(v * 0.7071067811865476))


def _fused_body(x_ref, pi_ref, W1_ref, b1_ref, W2_ref, b2_ref, Wd_ref, Wu_ref,
                o_ref, *, tm, kh, hid, r, lanes):
    f32 = jnp.float32
    x = x_ref[...]

    eidx = (jax.lax.broadcasted_iota(jnp.int32, (tm, lanes), 1) // r).astype(f32)
    p0 = pi_ref[:, 0:1]
    p1 = pi_ref[:, 1:2]
    i0 = pi_ref[:, 2:3]
    i1 = pi_ref[:, 3:4]
    w = (p0 * (i0 == eidx).astype(f32) + p1 * (i1 == eidx).astype(f32))

    h2 = _gelu_exact(jnp.dot(x, Wd_ref[...], preferred_element_type=f32))
    acc = jnp.dot(h2 * w, Wu_ref[...], preferred_element_type=f32)
    acc = acc + b2_ref[...]

    for k in range(hid // kh):
        h = _gelu_exact(
            jnp.dot(x, W1_ref[:, k * kh:(k + 1) * kh],
                    preferred_element_type=f32) + b1_ref[:, k * kh:(k + 1) * kh])
        acc = acc + jnp.dot(h, W2_ref[k * kh:(k + 1) * kh, :],
                            preferred_element_type=f32)
    o_ref[...] = acc


def kernel(x, gate, topk_probs, topk_idx, W1, b1, W2, b2, w_down, w_up):
    del gate
    orig_shape = x.shape
    dim = x.shape[-1]
    e, _, r = w_down.shape
    hid = W1.shape[1]
    alpha = 8.0
    scaling = alpha / r

    xf = x.reshape(-1, dim)
    t = xf.shape[0]
    tm = min(512, t)
    kh = min(512, hid)
    lanes = 128

    wd = jnp.transpose(w_down, (1, 0, 2)).reshape(dim, e * r)
    wd = jnp.pad(wd, ((0, 0), (0, lanes - e * r)))
    wu = w_up.reshape(e * r, dim) * scaling
    wu = jnp.pad(wu, ((0, lanes - e * r), (0, 0)))

    pi = jnp.concatenate(
        [topk_probs.astype(jnp.float32), topk_idx.astype(jnp.float32)], axis=1)

    b1r = b1.reshape(1, hid)
    b2r = b2.reshape(1, dim)

    grid = (t // tm,)
    out = pl.pallas_call(
        functools.partial(_fused_body, tm=tm, kh=kh, hid=hid, r=r,
                          lanes=lanes),
        grid=grid,
        in_specs=[
            pl.BlockSpec((tm, dim), lambda i: (i, 0)),
            pl.BlockSpec((tm, 4), lambda i: (i, 0)),
            pl.BlockSpec((dim, hid), lambda i: (0, 0)),
            pl.BlockSpec((1, hid), lambda i: (0, 0)),
            pl.BlockSpec((hid, dim), lambda i: (0, 0)),
            pl.BlockSpec((1, dim), lambda i: (0, 0)),
            pl.BlockSpec((dim, lanes), lambda i: (0, 0)),
            pl.BlockSpec((lanes, dim), lambda i: (0, 0)),
        ],
        out_specs=pl.BlockSpec((tm, dim), lambda i: (i, 0)),
        out_shape=jax.ShapeDtypeStruct((t, dim), jnp.float32),
    )(xf, pi, W1, b1r, W2, b2r, wd, wu)
    return out.reshape(orig_shape)
